# baseline (speedup 1.0000x reference)
import jax
import jax.numpy as jnp
from jax.experimental import pallas as pl
from jax.experimental.pallas import tpu as pltpu
from jax.sharding import NamedSharding, PartitionSpec as P

T, D_MODEL, D_FF, E, TOP_K = 32, 768, 1536, 8, 2
HM = D_MODEL // 2

_NDEV = 2 if jax.device_count() % 2 == 0 else 1
E_LOC = E // _NDEV
_MESH = jax.make_mesh((_NDEV,), ("x",))


def _moe_body(eoff_ref, x_ref, ids_ref, tw_ref, w1_ref, w3_ref, w2a_ref,
              w2b_ref, out_ref):
    e = pl.program_id(0)

    @pl.when(e == 0)
    def _init():
        out_ref[...] = jnp.zeros_like(out_ref)

    x = x_ref[...]
    h1 = jax.lax.dot_general(
        x, w1_ref[0, 0], (((1,), (1,)), ((), ())),
        preferred_element_type=jnp.float32)
    h3 = jax.lax.dot_general(
        x, w3_ref[0, 0], (((1,), (1,)), ((), ())),
        preferred_element_type=jnp.float32)
    act = h1 * jax.nn.sigmoid(h1) * h3
    oa = jax.lax.dot_general(
        act, w2a_ref[0, 0], (((1,), (1,)), ((), ())),
        preferred_element_type=jnp.float32)
    ob = jax.lax.dot_general(
        act, w2b_ref[0, 0], (((1,), (1,)), ((), ())),
        preferred_element_type=jnp.float32)

    gate = jnp.sum(
        jnp.where(ids_ref[...] == e + eoff_ref[0], tw_ref[...], 0.0),
        axis=1, keepdims=True)
    out_ref[:, :HM] += gate * oa
    out_ref[:, HM:] += gate * ob


def _local_moe(x, topk_ids, topk_weight, w13, w2):
    eoff = (jax.lax.axis_index("x") * E_LOC).astype(jnp.int32).reshape(1)
    part = pl.pallas_call(
        _moe_body,
        grid=(E_LOC,),
        in_specs=[
            pl.BlockSpec(memory_space=pltpu.SMEM),
            pl.BlockSpec((T, D_MODEL), lambda e: (0, 0)),
            pl.BlockSpec((T, TOP_K), lambda e: (0, 0)),
            pl.BlockSpec((T, TOP_K), lambda e: (0, 0)),
            pl.BlockSpec((1, 1, D_FF, D_MODEL), lambda e: (e, 0, 0, 0)),
            pl.BlockSpec((1, 1, D_FF, D_MODEL), lambda e: (e, 1, 0, 0)),
            pl.BlockSpec((1, 1, HM, D_FF), lambda e: (e, 0, 0, 0)),
            pl.BlockSpec((1, 1, HM, D_FF), lambda e: (e, 1, 0, 0)),
        ],
        out_specs=pl.BlockSpec((T, D_MODEL), lambda e: (0, 0)),
        out_shape=jax.ShapeDtypeStruct((T, D_MODEL), jnp.float32),
    )(eoff, x, topk_ids, topk_weight, w13, w13, w2, w2)
    return jax.lax.psum(part, "x")


@jax.jit
def kernel(x, topk_ids, topk_weight, w13_weight, w2_weight):
    w13 = w13_weight.reshape(E, 2, D_FF, D_MODEL)
    w2 = w2_weight.reshape(E, 2, HM, D_FF)
    w13 = jax.reshard(w13, NamedSharding(_MESH, P("x", None, None, None)))
    w2 = jax.reshard(w2, NamedSharding(_MESH, P("x", None, None, None)))
    return jax.shard_map(
        _local_moe,
        mesh=_MESH,
        check_vma=False,
        in_specs=(P(), P(), P(), P("x"), P("x")),
        out_specs=P(),
    )(x, topk_ids, topk_weight, w13, w2)


# speedup vs baseline: 8.8876x; 8.8876x over previous
import functools

import jax
import jax.numpy as jnp
from jax import lax
from jax.experimental import pallas as pl
from jax.experimental.pallas import tpu as pltpu
from jax.experimental.pallas import tpu_sc as plsc

T, D_MODEL, D_FF, E, TOP_K = 32, 768, 1536, 8, 2
HM = D_MODEL // 2
LANES = 16

_SC_MESH = plsc.VectorSubcoreMesh(core_axis_name="c", subcore_axis_name="s")


@functools.partial(
    pl.kernel,
    mesh=_SC_MESH,
    out_type=jax.ShapeDtypeStruct((E, T), jnp.float32),
    scratch_types=[
        pltpu.VMEM((TOP_K, T), jnp.int32),
        pltpu.VMEM((TOP_K, T), jnp.float32),
        pltpu.VMEM((E, T), jnp.float32),
    ],
)
def _gate_sc(ids_hbm, tw_hbm, out_hbm, ids_v, tw_v, out_v):
    wid = lax.axis_index("s") * 2 + lax.axis_index("c")

    @pl.when(wid == 0)
    def _():
        pltpu.sync_copy(ids_hbm, ids_v)
        pltpu.sync_copy(tw_hbm, tw_v)
        for e in range(E):
            for c in range(T // LANES):
                acc = jnp.zeros((LANES,), jnp.float32)
                for k in range(TOP_K):
                    idk = ids_v[k, pl.ds(c * LANES, LANES)]
                    twk = tw_v[k, pl.ds(c * LANES, LANES)]
                    acc = acc + jnp.where(idk == e, twk, 0.0)
                out_v[e, pl.ds(c * LANES, LANES)] = acc
        pltpu.sync_copy(out_v, out_hbm)


def _moe_body(g_ref, x_ref, w1_ref, w3_ref, w2a_ref, w2b_ref, out_ref):
    e = pl.program_id(0)

    @pl.when(e == 0)
    def _init():
        out_ref[...] = jnp.zeros_like(out_ref)

    x = x_ref[...]
    h1 = jax.lax.dot_general(
        x, w1_ref[0, 0], (((1,), (1,)), ((), ())),
        preferred_element_type=jnp.float32)
    h3 = jax.lax.dot_general(
        x, w3_ref[0, 0], (((1,), (1,)), ((), ())),
        preferred_element_type=jnp.float32)
    act = h1 * jax.nn.sigmoid(h1) * h3
    oa = jax.lax.dot_general(
        act, w2a_ref[0, 0], (((1,), (1,)), ((), ())),
        preferred_element_type=jnp.float32)
    ob = jax.lax.dot_general(
        act, w2b_ref[0, 0], (((1,), (1,)), ((), ())),
        preferred_element_type=jnp.float32)

    gate = jnp.transpose(g_ref[pl.ds(e, 1), :])
    out_ref[:, :HM] += gate * oa
    out_ref[:, HM:] += gate * ob


@jax.jit
def kernel(x, topk_ids, topk_weight, w13_weight, w2_weight):
    gates = _gate_sc(topk_ids.T, topk_weight.T)
    w13 = w13_weight.reshape(E, 2, D_FF, D_MODEL)
    w2 = w2_weight.reshape(E, 2, HM, D_FF)
    return pl.pallas_call(
        _moe_body,
        grid=(E,),
        in_specs=[
            pl.BlockSpec((E, T), lambda e: (0, 0)),
            pl.BlockSpec((T, D_MODEL), lambda e: (0, 0)),
            pl.BlockSpec((1, 1, D_FF, D_MODEL), lambda e: (e, 0, 0, 0)),
            pl.BlockSpec((1, 1, D_FF, D_MODEL), lambda e: (e, 1, 0, 0)),
            pl.BlockSpec((1, 1, HM, D_FF), lambda e: (e, 0, 0, 0)),
            pl.BlockSpec((1, 1, HM, D_FF), lambda e: (e, 1, 0, 0)),
        ],
        out_specs=pl.BlockSpec((T, D_MODEL), lambda e: (0, 0)),
        out_shape=jax.ShapeDtypeStruct((T, D_MODEL), jnp.float32),
    )(gates, x, w13, w13, w2, w2)


# speedup vs baseline: 13.3033x; 1.4968x over previous
import jax
import jax.numpy as jnp
from jax.experimental import pallas as pl

T, D_MODEL, D_FF, E, TOP_K = 32, 768, 1536, 8, 2


def _moe_body(x_ref, ids_ref, tw_ref, w1_ref, w3_ref, w2_ref, out_ref):
    e = pl.program_id(0)

    @pl.when(e == 0)
    def _init():
        out_ref[...] = jnp.zeros_like(out_ref)

    x = x_ref[...]
    h1 = jax.lax.dot_general(
        x, w1_ref[0, 0], (((1,), (1,)), ((), ())),
        preferred_element_type=jnp.float32)
    h3 = jax.lax.dot_general(
        x, w3_ref[0, 0], (((1,), (1,)), ((), ())),
        preferred_element_type=jnp.float32)
    act = h1 * jax.nn.sigmoid(h1) * h3
    o = jax.lax.dot_general(
        act, w2_ref[0], (((1,), (1,)), ((), ())),
        preferred_element_type=jnp.float32)

    gate = jnp.sum(
        jnp.where(ids_ref[...] == e, tw_ref[...], 0.0),
        axis=1, keepdims=True)
    out_ref[...] += gate * o


@jax.jit
def kernel(x, topk_ids, topk_weight, w13_weight, w2_weight):
    w13 = w13_weight.reshape(E, 2, D_FF, D_MODEL)
    return pl.pallas_call(
        _moe_body,
        grid=(E,),
        in_specs=[
            pl.BlockSpec((T, D_MODEL), lambda e: (0, 0)),
            pl.BlockSpec((T, TOP_K), lambda e: (0, 0)),
            pl.BlockSpec((T, TOP_K), lambda e: (0, 0)),
            pl.BlockSpec((1, 1, D_FF, D_MODEL), lambda e: (e, 0, 0, 0)),
            pl.BlockSpec((1, 1, D_FF, D_MODEL), lambda e: (e, 1, 0, 0)),
            pl.BlockSpec((1, D_MODEL, D_FF), lambda e: (e, 0, 0)),
        ],
        out_specs=pl.BlockSpec((T, D_MODEL), lambda e: (0, 0)),
        out_shape=jax.ShapeDtypeStruct((T, D_MODEL), jnp.float32),
    )(x, topk_ids, topk_weight, w13, w13, w2_weight)
